# baseline (speedup 1.0000x reference)
import functools

import jax
import jax.numpy as jnp
from jax import lax
from jax.experimental import pallas as pl
from jax.experimental.pallas import tpu as pltpu
from jax.experimental.pallas import tpu_sc as plsc

N = 10000
C = 128
H = 8
HC = H * C
HP = HC // 2
E = 160000
EP = 163840
EPR = EP // 8
AW = 16
GW = 128
NC = 2
NS = 16
NW = NC * NS
EPW = EP // NW
EPWR = EPW // 8
NP = 10240
RPT = NP // NS

KEA = 32
NITA = EPW // KEA
ICA = 8
KE2 = 128
NIT2 = EPW // KE2
KEB = 16
NITB = EPW // KEB
ICB = 16

_mesh = plsc.VectorSubcoreMesh(core_axis_name="c", subcore_axis_name="s")


def _proj_body(xs_ref, xd_ref, ws_ref, wd_ref, asf_ref, adf_ref,
               hs_ref, as_ref, ad_ref):
    xs = xs_ref[...]
    ws = ws_ref[...]
    hs = jnp.dot(xs, ws, preferred_element_type=jnp.float32)

    def rne16(x):
        xi = lax.bitcast_convert_type(x, jnp.int32)
        return ((xi + 0x7FFF + ((xi >> 16) & 1)) >> 16) & 0xFFFF

    packs = []
    for h in range(H):
        a = hs[:, h * C:h * C + C // 2]
        bb = hs[:, h * C + C // 2:(h + 1) * C]
        packs.append((rne16(bb) << 16) | rne16(a))
    hs_ref[...] = jnp.concatenate(packs, axis=1)

    rows = lax.broadcasted_iota(jnp.int32, (HC, GW), 0)
    cols = lax.broadcasted_iota(jnp.int32, (HC, GW), 1)
    sel = jnp.where(rows // C == cols, 1.0, 0.0).astype(jnp.float32)
    wsf = jnp.dot(ws * asf_ref[...], sel, preferred_element_type=jnp.float32)
    as_ref[...] = jnp.dot(xs, wsf, preferred_element_type=jnp.float32)
    wdf = jnp.dot(wd_ref[...] * adf_ref[...], sel,
                  preferred_element_type=jnp.float32)
    ad_ref[...] = jnp.dot(xd_ref[...], wdf, preferred_element_type=jnp.float32)


def _project(x_src, x_dst, Wsrc, Wdst, asrc, adst):
    nb = 1024
    return pl.pallas_call(
        _proj_body,
        grid=(NP // nb,),
        in_specs=[
            pl.BlockSpec((nb, C), lambda i: (i, 0)),
            pl.BlockSpec((nb, C), lambda i: (i, 0)),
            pl.BlockSpec((C, HC), lambda i: (0, 0)),
            pl.BlockSpec((C, HC), lambda i: (0, 0)),
            pl.BlockSpec((1, HC), lambda i: (0, 0)),
            pl.BlockSpec((1, HC), lambda i: (0, 0)),
        ],
        out_specs=[
            pl.BlockSpec((nb, HP), lambda i: (i, 0)),
            pl.BlockSpec((nb, GW), lambda i: (i, 0)),
            pl.BlockSpec((nb, GW), lambda i: (i, 0)),
        ],
        out_shape=[
            jax.ShapeDtypeStruct((NP, HP), jnp.int32),
            jax.ShapeDtypeStruct((NP, GW), jnp.float32),
            jax.ShapeDtypeStruct((NP, GW), jnp.float32),
        ],
    )(x_src, x_dst, Wsrc, Wdst, asrc.reshape(1, HC), adst.reshape(1, HC))


@functools.partial(
    pl.kernel,
    mesh=_mesh,
    out_type=(
        jax.ShapeDtypeStruct((EPR, GW), jnp.float32),
        jax.ShapeDtypeStruct((NC, NP, GW), jnp.float32),
    ),
    scratch_types=[
        pltpu.VMEM((ICA, 2 * KEA), jnp.int32),
        pltpu.VMEM((ICA, KEA), jnp.int32),
        pltpu.VMEM((2, 2 * KEA, GW), jnp.float32),
        pltpu.VMEM((ICA * KEA // 8, GW), jnp.float32),
        pltpu.VMEM((2, KEA, GW), jnp.float32),
        pltpu.VMEM_SHARED((NP, GW), jnp.float32),
        pltpu.SemaphoreType.DMA((2,)),
        pltpu.SemaphoreType.DMA((2,)),
    ],
)
def _phase_a(sad_hbm, dst_hbm, asd_hbm, ex_hbm, spart_hbm,
             idx_g, idx_d, agr, exb2, exw, s_acc, sem1, semsc):
    cid = lax.axis_index("c")
    sid = lax.axis_index("s")
    wid = sid * NC + cid

    def zbody(k, _):
        for bb in range(2):
            for j in range(GW // AW):
                exw[bb, k, pl.ds(j * AW, AW)] = jnp.zeros(
                    (AW,), jnp.float32)
        return 0
    lax.fori_loop(0, KEA, zbody, 0)

    def zcopy(t, _):
        pltpu.sync_copy(exw.at[0],
                        s_acc.at[pl.ds(sid * RPT + t * KEA, KEA)])
        return 0
    lax.fori_loop(0, RPT // KEA, zcopy, 0)
    plsc.subcore_barrier()

    def issue(ii, b):
        pltpu.async_copy(asd_hbm.at[idx_g.at[ii]], agr.at[b], sem1.at[b])

    def wait(ii, b):
        pltpu.make_async_copy(asd_hbm.at[idx_g.at[ii]], agr.at[b],
                              sem1.at[b]).wait()

    def outer(c, _):
        pltpu.sync_copy(sad_hbm.at[wid, pl.ds(c * ICA, ICA)], idx_g)
        pltpu.sync_copy(dst_hbm.at[wid, pl.ds(c * ICA, ICA)], idx_d)
        issue(0, 0)

        def inner(j, _):
            for b in range(2):
                ii = 2 * j + b
                wait(ii, b)

                @pl.when(ii < ICA - 1)
                def _():
                    issue(ii + 1, 1 - b)

                @pl.when(c * ICA + ii >= 2)
                def _():
                    pltpu.make_async_copy(exw.at[b], s_acc.at[idx_d.at[ii]],
                                          semsc.at[b]).wait()

                for k in range(KEA):
                    v = (agr.at[b][k, pl.ds(0, AW)]
                         + agr.at[b][KEA + k, pl.ds(0, AW)])
                    v = jnp.maximum(v, 0.2 * v)
                    v = jnp.exp(v)
                    exw[b, k, pl.ds(0, AW)] = v
                    exb2[ii * (KEA // 8) + k // 8,
                         pl.ds((k % 8) * AW, AW)] = v
                pltpu.async_copy(exw.at[b], s_acc.at[idx_d.at[ii]],
                                 semsc.at[b], add=True)
            return 0
        lax.fori_loop(0, ICA // 2, inner, 0)
        pltpu.sync_copy(exb2,
                        ex_hbm.at[pl.ds(wid * EPWR + c * (ICA * KEA // 8),
                                        ICA * KEA // 8)])
        return 0
    lax.fori_loop(0, NITA // ICA, outer, 0)

    for bb in range(2):
        pltpu.make_async_copy(exw.at[bb], s_acc.at[idx_d.at[bb]],
                              semsc.at[bb]).wait()
    plsc.subcore_barrier()

    def wcopy(t, _):
        r0 = sid * RPT + t * KEA
        pltpu.sync_copy(s_acc.at[pl.ds(r0, KEA)], exw.at[0])
        pltpu.sync_copy(exw.at[0], spart_hbm.at[cid, pl.ds(r0, KEA)])
        return 0
    lax.fori_loop(0, RPT // KEA, wcopy, 0)


def _ssum_body(p_ref, o_ref):
    o_ref[...] = p_ref[0] + p_ref[1]


def _ssum(parts):
    nb = 1024
    return pl.pallas_call(
        _ssum_body,
        grid=(NP // nb,),
        in_specs=[pl.BlockSpec((NC, nb, GW), lambda i: (0, i, 0))],
        out_specs=pl.BlockSpec((nb, GW), lambda i: (i, 0)),
        out_shape=jax.ShapeDtypeStruct((NP, GW), jnp.float32),
    )(parts)


@functools.partial(
    pl.kernel,
    mesh=_mesh,
    out_type=jax.ShapeDtypeStruct((EPR, GW), jnp.float32),
    scratch_types=[
        pltpu.VMEM((NIT2, KE2), jnp.int32),
        pltpu.VMEM((2, KE2, GW), jnp.float32),
        pltpu.VMEM((KE2 // 8, GW), jnp.float32),
        pltpu.VMEM((KE2 // 8, GW), jnp.float32),
        pltpu.SemaphoreType.DMA((2,)),
    ],
)
def _phase_a2(dst_hbm, ex_hbm, s_hbm, al_hbm,
              idx_d, ssr, exb, alb, sem):
    cid = lax.axis_index("c")
    sid = lax.axis_index("s")
    wid = sid * NC + cid

    pltpu.sync_copy(dst_hbm.at[wid], idx_d)
    pltpu.async_copy(s_hbm.at[idx_d.at[0]], ssr.at[0], sem.at[0])

    def body(j, _):
        for b in range(2):
            i = 2 * j + b

            @pl.when(i < NIT2 - 1)
            def _():
                pltpu.async_copy(s_hbm.at[idx_d.at[i + 1]], ssr.at[1 - b],
                                 sem.at[1 - b])
            pltpu.make_async_copy(s_hbm.at[idx_d.at[i]], ssr.at[b],
                                  sem.at[b]).wait()
            base = wid * EPWR + i * (KE2 // 8)
            pltpu.sync_copy(ex_hbm.at[pl.ds(base, KE2 // 8)], exb)
            for k in range(KE2):
                sv = ssr.at[b][k, pl.ds(0, AW)]
                ev = exb[k // 8, pl.ds((k % 8) * AW, AW)]
                alb[k // 8, pl.ds((k % 8) * AW, AW)] = (
                    ev * (1.0 / H) / (sv + 1e-16))
            pltpu.sync_copy(alb, al_hbm.at[pl.ds(base, KE2 // 8)])
        return 0
    lax.fori_loop(0, NIT2 // 2, body, 0)


@functools.partial(
    pl.kernel,
    mesh=_mesh,
    out_type=jax.ShapeDtypeStruct((NC, NP, C), jnp.float32),
    scratch_types=[
        pltpu.VMEM((ICB, KEB), jnp.int32),
        pltpu.VMEM((ICB // 2, 2 * KEB), jnp.int32),
        pltpu.VMEM((ICB * KEB // 8, GW), jnp.float32),
        pltpu.VMEM((2, KEB, HP), jnp.int32),
        pltpu.VMEM((2, 2 * KEB, C), jnp.float32),
        pltpu.VMEM_SHARED((NP, C), jnp.float32),
        pltpu.SemaphoreType.DMA((2,)),
        pltpu.SemaphoreType.DMA((2,)),
    ],
)
def _phase_b(src_hbm, dst_hbm, al_hbm, hs_hbm, opart_hbm,
             idx_s, idx_d, alb, hsb, msgb, out_acc, sem1, semsc):
    cid = lax.axis_index("c")
    sid = lax.axis_index("s")
    wid = sid * NC + cid

    def zbody(k, _):
        for bb in range(2):
            for j in range(C // AW):
                msgb[bb, k, pl.ds(j * AW, AW)] = jnp.zeros(
                    (AW,), jnp.float32)
        return 0
    lax.fori_loop(0, 2 * KEB, zbody, 0)

    def zcopy(t, _):
        pltpu.sync_copy(msgb.at[0],
                        out_acc.at[pl.ds(sid * RPT + t * 2 * KEB, 2 * KEB)])
        return 0
    lax.fori_loop(0, RPT // (2 * KEB), zcopy, 0)
    plsc.subcore_barrier()

    def issue(ii, b):
        pltpu.async_copy(hs_hbm.at[idx_s.at[ii]], hsb.at[b], sem1.at[b])

    def wait(ii, b):
        pltpu.make_async_copy(hs_hbm.at[idx_s.at[ii]], hsb.at[b],
                              sem1.at[b]).wait()

    def outer(c, _):
        pltpu.sync_copy(src_hbm.at[wid, pl.ds(c * ICB, ICB)], idx_s)
        pltpu.sync_copy(dst_hbm.at[wid, pl.ds(c * (ICB // 2), ICB // 2)],
                        idx_d)
        pltpu.sync_copy(
            al_hbm.at[pl.ds(wid * EPWR + c * (ICB * KEB // 8),
                            ICB * KEB // 8)], alb)
        issue(0, 0)

        def inner(jz, _):
            for jj in range(2):
                jp = 2 * jz + jj

                @pl.when(c * (ICB // 2) + jp >= 2)
                def _():
                    pltpu.make_async_copy(msgb.at[jj],
                                          out_acc.at[idx_d.at[jp]],
                                          semsc.at[jj]).wait()
                for b in range(2):
                    ii = 2 * jp + b
                    wait(ii, b)

                    @pl.when(ii < ICB - 1)
                    def _():
                        issue(ii + 1, 1 - b)

                    for k in range(KEB):
                        av = alb[ii * (KEB // 8) + k // 8,
                                 pl.ds((k % 8) * AW, AW)]
                        acca = [jnp.zeros((AW,), jnp.float32)
                                for _ in range(4)]
                        accb = [jnp.zeros((AW,), jnp.float32)
                                for _ in range(4)]
                        for h in range(H):
                            a = av[h]
                            for m in range(4):
                                vi = hsb.at[b][k,
                                               pl.ds(h * 64 + m * AW, AW)]
                                ua = lax.bitcast_convert_type(
                                    vi << 16, jnp.float32)
                                ub = lax.bitcast_convert_type(
                                    vi & jnp.int32(-65536), jnp.float32)
                                acca[m] = acca[m] + a * ua
                                accb[m] = accb[m] + a * ub
                        for m in range(4):
                            msgb[jj, b * KEB + k,
                                 pl.ds(m * AW, AW)] = acca[m]
                            msgb[jj, b * KEB + k,
                                 pl.ds(64 + m * AW, AW)] = accb[m]
                pltpu.async_copy(msgb.at[jj], out_acc.at[idx_d.at[jp]],
                                 semsc.at[jj], add=True)
            return 0
        lax.fori_loop(0, ICB // 4, inner, 0)
        return 0
    lax.fori_loop(0, NITB // ICB, outer, 0)

    for bb in range(2):
        pltpu.make_async_copy(msgb.at[bb], out_acc.at[idx_d.at[bb]],
                              semsc.at[bb]).wait()
    plsc.subcore_barrier()

    def wcopy(t, _):
        r0 = sid * RPT + t * 2 * KEB
        pltpu.sync_copy(out_acc.at[pl.ds(r0, 2 * KEB)], msgb.at[0])
        pltpu.sync_copy(msgb.at[0], opart_hbm.at[cid, pl.ds(r0, 2 * KEB)])
        return 0
    lax.fori_loop(0, RPT // (2 * KEB), wcopy, 0)


def _fin_body(p_ref, b_ref, w_ref, lb_ref, o_ref):
    t = p_ref[0] + p_ref[1] + b_ref[...]
    mu = jnp.mean(t, axis=-1, keepdims=True)
    var = jnp.mean((t - mu) * (t - mu), axis=-1, keepdims=True)
    y = (t - mu) / jnp.sqrt(var + 1e-5) * w_ref[...] + lb_ref[...]
    o_ref[...] = jnp.maximum(y, 0.0)


def _finalize(parts, b, lnw, lnb):
    nb = 1024
    return pl.pallas_call(
        _fin_body,
        grid=(NP // nb,),
        in_specs=[
            pl.BlockSpec((NC, nb, C), lambda i: (0, i, 0)),
            pl.BlockSpec((1, C), lambda i: (0, 0)),
            pl.BlockSpec((1, C), lambda i: (0, 0)),
            pl.BlockSpec((1, C), lambda i: (0, 0)),
        ],
        out_specs=pl.BlockSpec((nb, C), lambda i: (i, 0)),
        out_shape=jax.ShapeDtypeStruct((NP, C), jnp.float32),
    )(parts, b.reshape(1, C), lnw.reshape(1, C), lnb.reshape(1, C))


def _conv(x_src, x_dst, ei, Wsrc, Wdst, asrc, adst):
    sAd, dA, d2, sB, dB2 = ei
    hs, a_s, a_d = _project(x_src, x_dst, Wsrc, Wdst, asrc, adst)
    asd = jnp.concatenate([a_s, a_d], axis=0)
    ex, spart = _phase_a(sAd, dA, asd)
    s_sum = _ssum(spart)
    alpha = _phase_a2(d2, ex, s_sum)
    opart = _phase_b(sB, dB2, alpha, hs)
    return opart


def _pad_edges(ei):
    npad = EP - E
    src = jnp.concatenate(
        [ei[0], (jnp.arange(npad, dtype=jnp.int32) * 37) % N])
    dst = jnp.concatenate(
        [ei[1], N + (jnp.arange(npad, dtype=jnp.int32) % (NP - N))])
    src3a = src.reshape(NW, NITA, KEA)
    dst3a = dst.reshape(NW, NITA, KEA)
    sad = jnp.concatenate([src3a, dst3a + NP], axis=2)
    return (sad, dst3a,
            dst.reshape(NW, NIT2, KE2),
            src.reshape(NW, NITB, KEB),
            dst.reshape(NW, NITB // 2, 2 * KEB))


def kernel(x_user, x_item, edge_index_u2i, edge_index_i2u,
           Wsrc_0_u2i, Wdst_0_u2i, asrc_0_u2i, adst_0_u2i, b_0_u2i,
           Wsrc_0_i2u, Wdst_0_i2u, asrc_0_i2u, adst_0_i2u, b_0_i2u,
           lnw_0_user, lnb_0_user, lnw_0_item, lnb_0_item,
           Wsrc_1_u2i, Wdst_1_u2i, asrc_1_u2i, adst_1_u2i, b_1_u2i,
           Wsrc_1_i2u, Wdst_1_i2u, asrc_1_i2u, adst_1_i2u, b_1_i2u,
           lnw_1_user, lnb_1_user, lnw_1_item, lnb_1_item):
    p = dict(locals())
    ei_u2i = _pad_edges(edge_index_u2i)
    ei_i2u = _pad_edges(edge_index_i2u)
    zp = jnp.zeros((NP - N, C), jnp.float32)
    xu = jnp.concatenate([x_user, zp], axis=0)
    xi = jnp.concatenate([x_item, zp], axis=0)
    for l in range(2):
        op_i = _conv(xu, xi, ei_u2i,
                     p[f"Wsrc_{l}_u2i"], p[f"Wdst_{l}_u2i"],
                     p[f"asrc_{l}_u2i"], p[f"adst_{l}_u2i"])
        op_u = _conv(xi, xu, ei_i2u,
                     p[f"Wsrc_{l}_i2u"], p[f"Wdst_{l}_i2u"],
                     p[f"asrc_{l}_i2u"], p[f"adst_{l}_i2u"])
        xi = _finalize(op_i, p[f"b_{l}_u2i"], p[f"lnw_{l}_item"],
                       p[f"lnb_{l}_item"])
        xu = _finalize(op_u, p[f"b_{l}_i2u"], p[f"lnw_{l}_user"],
                       p[f"lnb_{l}_user"])
    return jnp.stack([xu[:N], xi[:N]], axis=0)


# speedup vs baseline: 1.1146x; 1.1146x over previous
import functools

import jax
import jax.numpy as jnp
from jax import lax
from jax.experimental import pallas as pl
from jax.experimental.pallas import tpu as pltpu
from jax.experimental.pallas import tpu_sc as plsc

N = 10000
C = 128
H = 8
HC = H * C
HP = HC // 2
E = 160000
EP = 163840
EPR = EP // 8
AW = 16
GW = 128
NC = 2
NS = 16
NW = NC * NS
EPW = EP // NW
EPWR = EPW // 8
NP = 10240
RPT = NP // NS

KEA = 32
NITA = EPW // KEA
ICA = 8
KE2 = 128
NIT2 = EPW // KE2
KEB = 16
NITB = EPW // KEB
ICB = 16

_mesh = plsc.VectorSubcoreMesh(core_axis_name="c", subcore_axis_name="s")


def _proj_body(xs_ref, xd_ref, ws_ref, wd_ref, asf_ref, adf_ref,
               hs_ref, as_ref, ad_ref):
    xs = xs_ref[...]
    ws = ws_ref[...]
    hs = jnp.dot(xs, ws, preferred_element_type=jnp.float32)

    def rne16(x):
        xi = lax.bitcast_convert_type(x, jnp.int32)
        return ((xi + 0x7FFF + ((xi >> 16) & 1)) >> 16) & 0xFFFF

    packs = []
    for h in range(H):
        a = hs[:, h * C:h * C + C // 2]
        bb = hs[:, h * C + C // 2:(h + 1) * C]
        packs.append((rne16(bb) << 16) | rne16(a))
    hs_ref[...] = jnp.concatenate(packs, axis=1)

    rows = lax.broadcasted_iota(jnp.int32, (HC, GW), 0)
    cols = lax.broadcasted_iota(jnp.int32, (HC, GW), 1)
    sel = jnp.where(rows // C == cols, 1.0, 0.0).astype(jnp.float32)
    wsf = jnp.dot(ws * asf_ref[...], sel, preferred_element_type=jnp.float32)
    as_ref[...] = jnp.dot(xs, wsf, preferred_element_type=jnp.float32)
    wdf = jnp.dot(wd_ref[...] * adf_ref[...], sel,
                  preferred_element_type=jnp.float32)
    ad_ref[...] = jnp.dot(xd_ref[...], wdf, preferred_element_type=jnp.float32)


def _project(x_src, x_dst, Wsrc, Wdst, asrc, adst):
    nb = 1024
    return pl.pallas_call(
        _proj_body,
        grid=(NP // nb,),
        in_specs=[
            pl.BlockSpec((nb, C), lambda i: (i, 0)),
            pl.BlockSpec((nb, C), lambda i: (i, 0)),
            pl.BlockSpec((C, HC), lambda i: (0, 0)),
            pl.BlockSpec((C, HC), lambda i: (0, 0)),
            pl.BlockSpec((1, HC), lambda i: (0, 0)),
            pl.BlockSpec((1, HC), lambda i: (0, 0)),
        ],
        out_specs=[
            pl.BlockSpec((nb, HP), lambda i: (i, 0)),
            pl.BlockSpec((nb, GW), lambda i: (i, 0)),
            pl.BlockSpec((nb, GW), lambda i: (i, 0)),
        ],
        out_shape=[
            jax.ShapeDtypeStruct((NP, HP), jnp.int32),
            jax.ShapeDtypeStruct((NP, GW), jnp.float32),
            jax.ShapeDtypeStruct((NP, GW), jnp.float32),
        ],
    )(x_src, x_dst, Wsrc, Wdst, asrc.reshape(1, HC), adst.reshape(1, HC))


@functools.partial(
    pl.kernel,
    mesh=_mesh,
    out_type=(
        jax.ShapeDtypeStruct((EPR, GW), jnp.float32),
        jax.ShapeDtypeStruct((NC, NP, GW), jnp.float32),
    ),
    scratch_types=[
        pltpu.VMEM((ICA, 2 * KEA), jnp.int32),
        pltpu.VMEM((ICA, KEA), jnp.int32),
        pltpu.VMEM((2, 2 * KEA, GW), jnp.float32),
        pltpu.VMEM((ICA * KEA // 8, GW), jnp.float32),
        pltpu.VMEM((KEA, GW), jnp.float32),
        pltpu.VMEM_SHARED((NP, GW), jnp.float32),
        pltpu.SemaphoreType.DMA((2,)),
    ],
)
def _phase_a(sad_hbm, dst_hbm, asd_hbm, ex_hbm, spart_hbm,
             idx_g, idx_d, agr, exb2, exw, s_acc, sem1):
    cid = lax.axis_index("c")
    sid = lax.axis_index("s")
    wid = sid * NC + cid

    def zbody(k, _):
        for j in range(GW // AW):
            exw[k, pl.ds(j * AW, AW)] = jnp.zeros((AW,), jnp.float32)
        return 0
    lax.fori_loop(0, KEA, zbody, 0)

    def zcopy(t, _):
        pltpu.sync_copy(exw, s_acc.at[pl.ds(sid * RPT + t * KEA, KEA)])
        return 0
    lax.fori_loop(0, RPT // KEA, zcopy, 0)
    plsc.subcore_barrier()

    def issue(ii, b):
        pltpu.async_copy(asd_hbm.at[idx_g.at[ii]], agr.at[b], sem1.at[b])

    def wait(ii, b):
        pltpu.make_async_copy(asd_hbm.at[idx_g.at[ii]], agr.at[b],
                              sem1.at[b]).wait()

    def outer(c, _):
        pltpu.sync_copy(sad_hbm.at[wid, pl.ds(c * ICA, ICA)], idx_g)
        pltpu.sync_copy(dst_hbm.at[wid, pl.ds(c * ICA, ICA)], idx_d)
        issue(0, 0)

        def inner(j, _):
            for b in range(2):
                ii = 2 * j + b
                wait(ii, b)

                @pl.when(ii < ICA - 1)
                def _():
                    issue(ii + 1, 1 - b)

                for k in range(KEA):
                    v = (agr.at[b][k, pl.ds(0, AW)]
                         + agr.at[b][KEA + k, pl.ds(0, AW)])
                    v = jnp.maximum(v, 0.2 * v)
                    v = jnp.exp(v)
                    exw[k, pl.ds(0, AW)] = v
                    exb2[ii * (KEA // 8) + k // 8,
                         pl.ds((k % 8) * AW, AW)] = v
                pltpu.sync_copy(exw, s_acc.at[idx_d.at[ii]], add=True)
            return 0
        lax.fori_loop(0, ICA // 2, inner, 0)
        pltpu.sync_copy(exb2,
                        ex_hbm.at[pl.ds(wid * EPWR + c * (ICA * KEA // 8),
                                        ICA * KEA // 8)])
        return 0
    lax.fori_loop(0, NITA // ICA, outer, 0)

    plsc.subcore_barrier()

    def wcopy(t, _):
        r0 = sid * RPT + t * KEA
        pltpu.sync_copy(s_acc.at[pl.ds(r0, KEA)], exw)
        pltpu.sync_copy(exw, spart_hbm.at[cid, pl.ds(r0, KEA)])
        return 0
    lax.fori_loop(0, RPT // KEA, wcopy, 0)


def _ssum_body(p_ref, o_ref):
    o_ref[...] = p_ref[0] + p_ref[1]


def _ssum(parts):
    nb = 1024
    return pl.pallas_call(
        _ssum_body,
        grid=(NP // nb,),
        in_specs=[pl.BlockSpec((NC, nb, GW), lambda i: (0, i, 0))],
        out_specs=pl.BlockSpec((nb, GW), lambda i: (i, 0)),
        out_shape=jax.ShapeDtypeStruct((NP, GW), jnp.float32),
    )(parts)


@functools.partial(
    pl.kernel,
    mesh=_mesh,
    out_type=jax.ShapeDtypeStruct((EPR, GW), jnp.float32),
    scratch_types=[
        pltpu.VMEM((NIT2, KE2), jnp.int32),
        pltpu.VMEM((2, KE2, GW), jnp.float32),
        pltpu.VMEM((KE2 // 8, GW), jnp.float32),
        pltpu.VMEM((KE2 // 8, GW), jnp.float32),
        pltpu.SemaphoreType.DMA((2,)),
    ],
)
def _phase_a2(dst_hbm, ex_hbm, s_hbm, al_hbm,
              idx_d, ssr, exb, alb, sem):
    cid = lax.axis_index("c")
    sid = lax.axis_index("s")
    wid = sid * NC + cid

    pltpu.sync_copy(dst_hbm.at[wid], idx_d)
    pltpu.async_copy(s_hbm.at[idx_d.at[0]], ssr.at[0], sem.at[0])

    def body(j, _):
        for b in range(2):
            i = 2 * j + b

            @pl.when(i < NIT2 - 1)
            def _():
                pltpu.async_copy(s_hbm.at[idx_d.at[i + 1]], ssr.at[1 - b],
                                 sem.at[1 - b])
            pltpu.make_async_copy(s_hbm.at[idx_d.at[i]], ssr.at[b],
                                  sem.at[b]).wait()
            base = wid * EPWR + i * (KE2 // 8)
            pltpu.sync_copy(ex_hbm.at[pl.ds(base, KE2 // 8)], exb)
            for k in range(KE2):
                sv = ssr.at[b][k, pl.ds(0, AW)]
                ev = exb[k // 8, pl.ds((k % 8) * AW, AW)]
                alb[k // 8, pl.ds((k % 8) * AW, AW)] = (
                    ev * (1.0 / H) / (sv + 1e-16))
            pltpu.sync_copy(alb, al_hbm.at[pl.ds(base, KE2 // 8)])
        return 0
    lax.fori_loop(0, NIT2 // 2, body, 0)


@functools.partial(
    pl.kernel,
    mesh=_mesh,
    out_type=jax.ShapeDtypeStruct((NC, NP, C), jnp.float32),
    scratch_types=[
        pltpu.VMEM((ICB, KEB), jnp.int32),
        pltpu.VMEM((ICB // 2, 2 * KEB), jnp.int32),
        pltpu.VMEM((ICB * KEB // 8, GW), jnp.float32),
        pltpu.VMEM((2, KEB, HP), jnp.int32),
        pltpu.VMEM((2 * KEB, C), jnp.float32),
        pltpu.VMEM_SHARED((NP, C), jnp.float32),
        pltpu.SemaphoreType.DMA((2,)),
    ],
)
def _phase_b(src_hbm, dst_hbm, al_hbm, hs_hbm, opart_hbm,
             idx_s, idx_d, alb, hsb, msgb, out_acc, sem1):
    cid = lax.axis_index("c")
    sid = lax.axis_index("s")
    wid = sid * NC + cid

    def zbody(k, _):
        for j in range(C // AW):
            msgb[k, pl.ds(j * AW, AW)] = jnp.zeros((AW,), jnp.float32)
        return 0
    lax.fori_loop(0, 2 * KEB, zbody, 0)

    def zcopy(t, _):
        pltpu.sync_copy(msgb,
                        out_acc.at[pl.ds(sid * RPT + t * 2 * KEB, 2 * KEB)])
        return 0
    lax.fori_loop(0, RPT // (2 * KEB), zcopy, 0)
    plsc.subcore_barrier()

    def issue(ii, b):
        pltpu.async_copy(hs_hbm.at[idx_s.at[ii]], hsb.at[b], sem1.at[b])

    def wait(ii, b):
        pltpu.make_async_copy(hs_hbm.at[idx_s.at[ii]], hsb.at[b],
                              sem1.at[b]).wait()

    def outer(c, _):
        pltpu.sync_copy(src_hbm.at[wid, pl.ds(c * ICB, ICB)], idx_s)
        pltpu.sync_copy(dst_hbm.at[wid, pl.ds(c * (ICB // 2), ICB // 2)],
                        idx_d)
        pltpu.sync_copy(
            al_hbm.at[pl.ds(wid * EPWR + c * (ICB * KEB // 8),
                            ICB * KEB // 8)], alb)
        issue(0, 0)

        def inner(j, _):
            for b in range(2):
                ii = 2 * j + b
                wait(ii, b)

                @pl.when(ii < ICB - 1)
                def _():
                    issue(ii + 1, 1 - b)

                for k in range(KEB):
                    av = alb[ii * (KEB // 8) + k // 8,
                             pl.ds((k % 8) * AW, AW)]
                    acca = [jnp.zeros((AW,), jnp.float32) for _ in range(4)]
                    accb = [jnp.zeros((AW,), jnp.float32) for _ in range(4)]
                    for h in range(H):
                        a = av[h]
                        for m in range(4):
                            vi = hsb.at[b][k, pl.ds(h * 64 + m * AW, AW)]
                            ua = lax.bitcast_convert_type(
                                vi << 16, jnp.float32)
                            ub = lax.bitcast_convert_type(
                                vi & jnp.int32(-65536), jnp.float32)
                            acca[m] = acca[m] + a * ua
                            accb[m] = accb[m] + a * ub
                    for m in range(4):
                        msgb[b * KEB + k, pl.ds(m * AW, AW)] = acca[m]
                        msgb[b * KEB + k, pl.ds(64 + m * AW, AW)] = accb[m]
            pltpu.sync_copy(msgb, out_acc.at[idx_d.at[j]], add=True)
            return 0
        lax.fori_loop(0, ICB // 2, inner, 0)
        return 0
    lax.fori_loop(0, NITB // ICB, outer, 0)

    plsc.subcore_barrier()

    def wcopy(t, _):
        r0 = sid * RPT + t * 2 * KEB
        pltpu.sync_copy(out_acc.at[pl.ds(r0, 2 * KEB)], msgb)
        pltpu.sync_copy(msgb, opart_hbm.at[cid, pl.ds(r0, 2 * KEB)])
        return 0
    lax.fori_loop(0, RPT // (2 * KEB), wcopy, 0)


def _fin_body(p_ref, b_ref, w_ref, lb_ref, o_ref):
    t = p_ref[0] + p_ref[1] + b_ref[...]
    mu = jnp.mean(t, axis=-1, keepdims=True)
    var = jnp.mean((t - mu) * (t - mu), axis=-1, keepdims=True)
    y = (t - mu) / jnp.sqrt(var + 1e-5) * w_ref[...] + lb_ref[...]
    o_ref[...] = jnp.maximum(y, 0.0)


def _finalize(parts, b, lnw, lnb):
    nb = 1024
    return pl.pallas_call(
        _fin_body,
        grid=(NP // nb,),
        in_specs=[
            pl.BlockSpec((NC, nb, C), lambda i: (0, i, 0)),
            pl.BlockSpec((1, C), lambda i: (0, 0)),
            pl.BlockSpec((1, C), lambda i: (0, 0)),
            pl.BlockSpec((1, C), lambda i: (0, 0)),
        ],
        out_specs=pl.BlockSpec((nb, C), lambda i: (i, 0)),
        out_shape=jax.ShapeDtypeStruct((NP, C), jnp.float32),
    )(parts, b.reshape(1, C), lnw.reshape(1, C), lnb.reshape(1, C))


def _conv(x_src, x_dst, ei, Wsrc, Wdst, asrc, adst):
    sAd, dA, d2, sB, dB2 = ei
    hs, a_s, a_d = _project(x_src, x_dst, Wsrc, Wdst, asrc, adst)
    asd = jnp.concatenate([a_s, a_d], axis=0)
    ex, spart = _phase_a(sAd, dA, asd)
    s_sum = _ssum(spart)
    alpha = _phase_a2(d2, ex, s_sum)
    opart = _phase_b(sB, dB2, alpha, hs)
    return opart


def _pad_edges(ei):
    npad = EP - E
    src = jnp.concatenate(
        [ei[0], (jnp.arange(npad, dtype=jnp.int32) * 37) % N])
    dst = jnp.concatenate(
        [ei[1], N + (jnp.arange(npad, dtype=jnp.int32) % (NP - N))])
    src3a = src.reshape(NW, NITA, KEA)
    dst3a = dst.reshape(NW, NITA, KEA)
    sad = jnp.concatenate([src3a, dst3a + NP], axis=2)
    return (sad, dst3a,
            dst.reshape(NW, NIT2, KE2),
            src.reshape(NW, NITB, KEB),
            dst.reshape(NW, NITB // 2, 2 * KEB))


def kernel(x_user, x_item, edge_index_u2i, edge_index_i2u,
           Wsrc_0_u2i, Wdst_0_u2i, asrc_0_u2i, adst_0_u2i, b_0_u2i,
           Wsrc_0_i2u, Wdst_0_i2u, asrc_0_i2u, adst_0_i2u, b_0_i2u,
           lnw_0_user, lnb_0_user, lnw_0_item, lnb_0_item,
           Wsrc_1_u2i, Wdst_1_u2i, asrc_1_u2i, adst_1_u2i, b_1_u2i,
           Wsrc_1_i2u, Wdst_1_i2u, asrc_1_i2u, adst_1_i2u, b_1_i2u,
           lnw_1_user, lnb_1_user, lnw_1_item, lnb_1_item):
    p = dict(locals())
    ei_u2i = _pad_edges(edge_index_u2i)
    ei_i2u = _pad_edges(edge_index_i2u)
    zp = jnp.zeros((NP - N, C), jnp.float32)
    xu = jnp.concatenate([x_user, zp], axis=0)
    xi = jnp.concatenate([x_item, zp], axis=0)
    for l in range(2):
        op_i = _conv(xu, xi, ei_u2i,
                     p[f"Wsrc_{l}_u2i"], p[f"Wdst_{l}_u2i"],
                     p[f"asrc_{l}_u2i"], p[f"adst_{l}_u2i"])
        op_u = _conv(xi, xu, ei_i2u,
                     p[f"Wsrc_{l}_i2u"], p[f"Wdst_{l}_i2u"],
                     p[f"asrc_{l}_i2u"], p[f"adst_{l}_i2u"])
        xi = _finalize(op_i, p[f"b_{l}_u2i"], p[f"lnw_{l}_item"],
                       p[f"lnb_{l}_item"])
        xu = _finalize(op_u, p[f"b_{l}_i2u"], p[f"lnw_{l}_user"],
                       p[f"lnb_{l}_user"])
    return jnp.stack([xu[:N], xi[:N]], axis=0)
